# baseline (speedup 1.0000x reference)
import jax
import jax.numpy as jnp
from jax import lax
from jax.experimental import pallas as pl
from jax.experimental.pallas import tpu as pltpu
from jax.experimental.pallas import tpu_sc as plsc

_VOCAB = 100000
_EMBED = 64
_EPAD = 128
_CTX = 20
_BATCH = 1024
_HIDDEN = 128
_KAUG = 136

_NC = 2
_NS = 16
_NW = _NC * _NS
_BTOT = _BATCH * _CTX
_BPW = _BTOT // _NW
_CH = 128
_NCH = _BPW // _CH

_VB = 2048
_NJ = (_VOCAB + _VB - 1) // _VB
_VPAD = _NJ * _VB
_VTAIL = _VOCAB - (_NJ - 1) * _VB


def _gather_body(table_hbm, idx_hbm, out_hbm, idx_v, rows_v, sem):
    wid = lax.axis_index("s") * _NC + lax.axis_index("c")
    base = wid * _BPW
    pltpu.sync_copy(idx_hbm.at[pl.ds(base, _BPW)], idx_v)
    copies = []
    for i in range(_NCH):
        c = pltpu.make_async_copy(
            table_hbm.at[idx_v.at[pl.ds(i * _CH, _CH)]],
            rows_v.at[pl.ds(i * _CH, _CH)],
            sem,
        )
        c.start()
        copies.append(c)
    for c in copies:
        c.wait()
    pltpu.sync_copy(rows_v, out_hbm.at[pl.ds(base, _BPW)])


def _sc_gather(table128, idx):
    mesh = plsc.VectorSubcoreMesh(
        core_axis_name="c", subcore_axis_name="s",
        num_cores=_NC, num_subcores=_NS,
    )
    return pl.kernel(
        _gather_body,
        out_type=jax.ShapeDtypeStruct((_BTOT, _EPAD), jnp.float32),
        mesh=mesh,
        scratch_types=[
            pltpu.VMEM((_BPW,), jnp.int32),
            pltpu.VMEM((_BPW, _EPAD), jnp.float32),
            pltpu.SemaphoreType.DMA,
        ],
    )(table128, idx)


_NH = 2
_BH = _BATCH // _NH


def _mlp_body(embT, W1T, b1c, W2aT, out, h_ref, m_ref, s_ref):
    p = pl.program_id(1)
    j = pl.program_id(2)

    @pl.when((p == 0) & (j == 0))
    def _():
        pre = jnp.dot(W1T[...], embT[...], preferred_element_type=jnp.float32)
        pre = pre + b1c[...]
        h_ref[0:_HIDDEN, :] = jnp.maximum(pre, 0.0).astype(jnp.bfloat16)
        row = lax.broadcasted_iota(jnp.int32, (8, _BH), 0)
        ext = jnp.where(row == 0, 1.0, 0.0)
        h_ref[_HIDDEN:_KAUG, :] = ext.astype(jnp.bfloat16)
        m_ref[...] = jnp.full(m_ref.shape, -jnp.inf, jnp.float32)
        s_ref[...] = jnp.zeros(s_ref.shape, jnp.float32)

    @pl.when((p == 1) & (j == 0))
    def _():
        lse = m_ref[...] + jnp.log(s_ref[...])
        hi = lse.astype(jnp.bfloat16)
        lo = (lse - hi.astype(jnp.float32)).astype(jnp.bfloat16)
        h_ref[_HIDDEN + 1:_HIDDEN + 3, :] = jnp.concatenate([hi, lo], axis=0)

    logits = lax.dot_general(
        W2aT[...], h_ref[...],
        dimension_numbers=(((0,), (0,)), ((), ())),
        preferred_element_type=jnp.float32)

    @pl.when(p == 0)
    def _():
        bm = jnp.max(logits, axis=0, keepdims=True)
        new_m = jnp.maximum(m_ref[...], bm)
        s_ref[...] = s_ref[...] * jnp.exp(m_ref[...] - new_m) + jnp.sum(
            jnp.exp(logits - new_m), axis=0, keepdims=True)
        m_ref[...] = new_m

    @pl.when(p == 1)
    def _():
        out[...] = logits


def _mlp_logsoftmax_t(embT, W1T, b1c, W2aT):
    return pl.pallas_call(
        _mlp_body,
        grid=(_NH, 2, _NJ),
        in_specs=[
            pl.BlockSpec((_CTX * _EPAD, _BH), lambda h, p, j: (0, h)),
            pl.BlockSpec((_HIDDEN, _CTX * _EPAD), lambda h, p, j: (0, 0)),
            pl.BlockSpec((_HIDDEN, 1), lambda h, p, j: (0, 0)),
            pl.BlockSpec((_KAUG, _VB), lambda h, p, j: (0, j)),
        ],
        out_specs=pl.BlockSpec((_VB, _BH), lambda h, p, j: (j * p, h)),
        out_shape=jax.ShapeDtypeStruct((_VOCAB, _BATCH), jnp.float32),
        scratch_shapes=[
            pltpu.VMEM((_KAUG, _BH), jnp.bfloat16),
            pltpu.VMEM((1, _BH), jnp.float32),
            pltpu.VMEM((1, _BH), jnp.float32),
        ],
    )(embT, W1T, b1c, W2aT)


def _augment_w2_t(W2, b2):
    npad = _VPAD - _VOCAB
    w2b = jnp.pad(W2.astype(jnp.bfloat16), ((0, 0), (0, npad)))
    b2row = jnp.pad(b2.reshape(1, _VOCAB).astype(jnp.bfloat16),
                    ((0, 0), (0, npad)), constant_values=-1e30)
    ones2 = jnp.full((2, _VPAD), -1.0, jnp.bfloat16)
    zer5 = jnp.zeros((_KAUG - _HIDDEN - 3, _VPAD), jnp.bfloat16)
    return jnp.concatenate([w2b, b2row, ones2, zer5], axis=0)


def _widen_w1_t(W1):
    w1 = W1.astype(jnp.bfloat16).reshape(_CTX, _EMBED, _HIDDEN)
    w1 = jnp.pad(w1, ((0, 0), (0, _EPAD - _EMBED), (0, 0)))
    return w1.reshape(_CTX * _EPAD, _HIDDEN).T


def kernel(inputs, emb_table, W1, b1, W2, b2):
    idx = inputs.reshape(_BTOT)
    table128 = jnp.pad(emb_table, ((0, 0), (0, _EPAD - _EMBED)))
    embeds = _sc_gather(table128, idx)
    embT = embeds.astype(jnp.bfloat16).reshape(_BATCH, _CTX * _EPAD).T
    outT = _mlp_logsoftmax_t(
        embT,
        _widen_w1_t(W1),
        b1.reshape(_HIDDEN, 1),
        _augment_w2_t(W2, b2),
    )
    return outT.T


# speedup vs baseline: 1.1286x; 1.1286x over previous
import jax
import jax.numpy as jnp
from jax import lax
from jax.experimental import pallas as pl
from jax.experimental.pallas import tpu as pltpu
from jax.experimental.pallas import tpu_sc as plsc

_VOCAB = 100000
_EMBED = 64
_EPAD = 128
_CTX = 20
_BATCH = 1024
_HIDDEN = 128
_KAUG = 136

_NC = 2
_NS = 16
_NW = _NC * _NS
_BTOT = _BATCH * _CTX
_BPW = _BTOT // _NW
_CH = 128
_NCH = _BPW // _CH

_VB = 2048
_NJ = (_VOCAB + _VB - 1) // _VB
_VPAD = _NJ * _VB


def _gather_body(table_hbm, idx_hbm, out_hbm, idx_v, rows_v, sem):
    wid = lax.axis_index("s") * _NC + lax.axis_index("c")
    base = wid * _BPW
    pltpu.sync_copy(idx_hbm.at[pl.ds(base, _BPW)], idx_v)
    copies = []
    for i in range(_NCH):
        c = pltpu.make_async_copy(
            table_hbm.at[idx_v.at[pl.ds(i * _CH, _CH)]],
            rows_v.at[pl.ds(i * _CH, _CH)],
            sem,
        )
        c.start()
        copies.append(c)
    for c in copies:
        c.wait()
    pltpu.sync_copy(rows_v, out_hbm.at[pl.ds(base, _BPW)])


def _sc_gather(table128, idx):
    mesh = plsc.VectorSubcoreMesh(
        core_axis_name="c", subcore_axis_name="s",
        num_cores=_NC, num_subcores=_NS,
    )
    return pl.kernel(
        _gather_body,
        out_type=jax.ShapeDtypeStruct((_BTOT, _EPAD), jnp.float32),
        mesh=mesh,
        scratch_types=[
            pltpu.VMEM((_BPW,), jnp.int32),
            pltpu.VMEM((_BPW, _EPAD), jnp.float32),
            pltpu.SemaphoreType.DMA,
        ],
    )(table128, idx)


def _mlp_body(embT, W1T, b1c, W2aT, out, h_ref, m_ref, s_ref):
    p = pl.program_id(0)
    j = pl.program_id(1)

    @pl.when((p == 0) & (j == 0))
    def _():
        pre = jnp.dot(W1T[...], embT[...], preferred_element_type=jnp.float32)
        pre = pre + b1c[...]
        h_ref[0:_HIDDEN, :] = jnp.maximum(pre, 0.0).astype(jnp.bfloat16)
        row = lax.broadcasted_iota(jnp.int32, (8, _BATCH), 0)
        ext = jnp.where(row == 0, 1.0, 0.0)
        h_ref[_HIDDEN:_KAUG, :] = ext.astype(jnp.bfloat16)
        m_ref[...] = jnp.full(m_ref.shape, -jnp.inf, jnp.float32)
        s_ref[...] = jnp.zeros(s_ref.shape, jnp.float32)

    @pl.when((p == 1) & (j == 0))
    def _():
        lse = m_ref[...] + jnp.log(s_ref[...])
        hi = lse.astype(jnp.bfloat16)
        lo = (lse - hi.astype(jnp.float32)).astype(jnp.bfloat16)
        h_ref[_HIDDEN + 1:_HIDDEN + 3, :] = jnp.concatenate([hi, lo], axis=0)

    logits = lax.dot_general(
        W2aT[...], h_ref[...],
        dimension_numbers=(((0,), (0,)), ((), ())),
        preferred_element_type=jnp.float32)

    @pl.when(p == 0)
    def _():
        bm = jnp.max(logits, axis=0, keepdims=True)
        new_m = jnp.maximum(m_ref[...], bm)
        s_ref[...] = s_ref[...] * jnp.exp(m_ref[...] - new_m) + jnp.sum(
            jnp.exp(logits - new_m), axis=0, keepdims=True)
        m_ref[...] = new_m

    @pl.when(p == 1)
    def _():
        out[...] = logits


def _mlp_logsoftmax_t(embT, W1T, b1c, W2aT):
    return pl.pallas_call(
        _mlp_body,
        grid=(2, _NJ),
        in_specs=[
            pl.BlockSpec((_CTX * _EPAD, _BATCH), lambda p, j: (0, 0)),
            pl.BlockSpec((_HIDDEN, _CTX * _EPAD), lambda p, j: (0, 0)),
            pl.BlockSpec((_HIDDEN, 1), lambda p, j: (0, 0)),
            pl.BlockSpec((_KAUG, _VB), lambda p, j: (0, j)),
        ],
        out_specs=pl.BlockSpec((_VB, _BATCH), lambda p, j: (j * p, 0)),
        out_shape=jax.ShapeDtypeStruct((_VOCAB, _BATCH), jnp.float32),
        scratch_shapes=[
            pltpu.VMEM((_KAUG, _BATCH), jnp.bfloat16),
            pltpu.VMEM((1, _BATCH), jnp.float32),
            pltpu.VMEM((1, _BATCH), jnp.float32),
        ],
    )(embT, W1T, b1c, W2aT)


def _augment_w2_t(W2, b2):
    npad = _VPAD - _VOCAB
    w2b = jnp.pad(W2.astype(jnp.bfloat16), ((0, 0), (0, npad)))
    b2row = jnp.pad(b2.reshape(1, _VOCAB).astype(jnp.bfloat16),
                    ((0, 0), (0, npad)), constant_values=-1e30)
    ones2 = jnp.full((2, _VPAD), -1.0, jnp.bfloat16)
    zer5 = jnp.zeros((_KAUG - _HIDDEN - 3, _VPAD), jnp.bfloat16)
    return jnp.concatenate([w2b, b2row, ones2, zer5], axis=0)


def _widen_w1_t(W1):
    w1 = W1.astype(jnp.bfloat16).reshape(_CTX, _EMBED, _HIDDEN)
    w1 = jnp.pad(w1, ((0, 0), (0, _EPAD - _EMBED), (0, 0)))
    return w1.reshape(_CTX * _EPAD, _HIDDEN).T


def kernel(inputs, emb_table, W1, b1, W2, b2):
    idx = inputs.reshape(_BTOT)
    table128 = jnp.pad(emb_table, ((0, 0), (0, _EPAD - _EMBED)))
    embeds = _sc_gather(table128, idx)
    embT = embeds.astype(jnp.bfloat16).reshape(_BATCH, _CTX * _EPAD).T
    outT = _mlp_logsoftmax_t(
        embT,
        _widen_w1_t(W1),
        b1.reshape(_HIDDEN, 1),
        _augment_w2_t(W2, b2),
    )
    return outT.T
